# Initial kernel scaffold; baseline (speedup 1.0000x reference)
import jax
import jax.numpy as jnp
from jax.experimental import pallas as pl


def kernel(node_attrs, edge_index, edge_embedding, edge_lengths, pos, batch, W1, b1, W2, b2, W3, b3):
    raise NotImplementedError("write your pallas kernel here")


# speedup vs baseline: 2.8018x; 2.8018x over previous
import functools

import jax
import jax.numpy as jnp
from jax import lax
from jax.experimental import pallas as pl
from jax.experimental.pallas import tpu as pltpu
from jax.experimental.pallas import tpu_sc as plsc

N = 10000
E = 320000
D = 128
DE = 16
H = 128
G = 32

NC, NS = 2, 16
NW = NC * NS
CHUNK = 128
NCHUNKS = E // CHUNK
N_PAD = 10240
NROW = N_PAD // 128

MLP_BLK = 2000
PROJ_BLK = 1000

_f32 = jnp.float32


def _proj_body(na_ref, wa_ref, wb_ref, pa_ref, pb_ref):
    x = na_ref[...]
    pa_ref[...] = lax.dot_general(x, wa_ref[...], (((1,), (0,)), ((), ())),
                                  preferred_element_type=_f32)
    pb_ref[...] = lax.dot_general(x, wb_ref[...], (((1,), (0,)), ((), ())),
                                  preferred_element_type=_f32)


def _node_proj(node_attrs, w1a, w1b):
    grid = N // PROJ_BLK
    return pl.pallas_call(
        _proj_body,
        grid=(grid,),
        in_specs=[
            pl.BlockSpec((PROJ_BLK, D), lambda i: (i, 0)),
            pl.BlockSpec((D, H), lambda i: (0, 0)),
            pl.BlockSpec((D, H), lambda i: (0, 0)),
        ],
        out_specs=[
            pl.BlockSpec((PROJ_BLK, H), lambda i: (i, 0)),
            pl.BlockSpec((PROJ_BLK, H), lambda i: (i, 0)),
        ],
        out_shape=[
            jax.ShapeDtypeStruct((N, H), _f32),
            jax.ShapeDtypeStruct((N, H), _f32),
        ],
    )(node_attrs, w1a, w1b)


def _gather_body(pa_hbm, pb_hbm, ec_hbm, en_hbm, x_hbm,
                 idx_c, idx_n, rows_a, rows_b, sem_a, sem_b):
    cid = lax.axis_index("c")
    sid = lax.axis_index("s")
    wid = sid * NC + cid
    nch = (NCHUNKS // NW) + jnp.where(wid < (NCHUNKS % NW), 1, 0)

    def body(i, carry):
        k = wid + i * NW
        base = k * CHUNK
        pltpu.sync_copy(ec_hbm.at[pl.ds(base, CHUNK)], idx_c)
        pltpu.sync_copy(en_hbm.at[pl.ds(base, CHUNK)], idx_n)
        ca = pltpu.async_copy(pa_hbm.at[idx_c], rows_a, sem_a)
        cb = pltpu.async_copy(pb_hbm.at[idx_n], rows_b, sem_b)
        ca.wait()
        cb.wait()

        def row_body(r, c2):
            for c in range(H // 16):
                sl = pl.ds(c * 16, 16)
                rows_a[r, sl] = rows_a[r, sl] + rows_b[r, sl]
            return c2
        lax.fori_loop(0, CHUNK, row_body, 0)
        pltpu.sync_copy(rows_a, x_hbm.at[pl.ds(base, CHUNK)])
        return carry

    lax.fori_loop(0, nch, body, 0)


def _edge_gather(pa, pb, ec, en):
    mesh = plsc.VectorSubcoreMesh(core_axis_name="c", subcore_axis_name="s")
    f = functools.partial(
        pl.kernel,
        out_type=jax.ShapeDtypeStruct((E, H), _f32),
        mesh=mesh,
        compiler_params=pltpu.CompilerParams(needs_layout_passes=False),
        scratch_types=[
            pltpu.VMEM((CHUNK,), jnp.int32),
            pltpu.VMEM((CHUNK,), jnp.int32),
            pltpu.VMEM((CHUNK, H), _f32),
            pltpu.VMEM((CHUNK, H), _f32),
            pltpu.SemaphoreType.DMA,
            pltpu.SemaphoreType.DMA,
        ],
    )(_gather_body)
    return f(pa, pb, ec, en)


def _mlp_body(x_ref, emb_ref, w1e_ref, b1_ref, w2_ref, b2_ref, w3_ref, b3_ref,
              out_ref):
    h = x_ref[...] + lax.dot_general(
        emb_ref[...], w1e_ref[...], (((1,), (0,)), ((), ())),
        preferred_element_type=_f32) + b1_ref[...]
    h = h * jax.nn.sigmoid(h)
    h = lax.dot_general(h, w2_ref[...], (((1,), (0,)), ((), ())),
                        preferred_element_type=_f32) + b2_ref[...]
    h = h * jax.nn.sigmoid(h)
    out_ref[...] = lax.dot_general(h, w3_ref[...], (((1,), (0,)), ((), ())),
                                   preferred_element_type=_f32) + b3_ref[...]


def _edge_mlp(x, emb, w1e, b1, w2, b2, w3, b3):
    grid = E // MLP_BLK
    return pl.pallas_call(
        _mlp_body,
        grid=(grid,),
        in_specs=[
            pl.BlockSpec((MLP_BLK, H), lambda i: (i, 0)),
            pl.BlockSpec((MLP_BLK, DE), lambda i: (i, 0)),
            pl.BlockSpec((DE, H), lambda i: (0, 0)),
            pl.BlockSpec((1, H), lambda i: (0, 0)),
            pl.BlockSpec((H, H), lambda i: (0, 0)),
            pl.BlockSpec((1, H), lambda i: (0, 0)),
            pl.BlockSpec((H, 1), lambda i: (0, 0)),
            pl.BlockSpec((1, 1), lambda i: (0, 0)),
        ],
        out_specs=pl.BlockSpec((MLP_BLK, 1), lambda i: (i, 0)),
        out_shape=jax.ShapeDtypeStruct((E, 1), _f32),
    )(x, emb, w1e, b1, w2, b2, w3, b3)


SC_CH = 2000
PER_W = E // NW


def _scatter_body(ch_hbm, ec_hbm, parts_hbm, vals, idxs, nacc, sem):
    cid = lax.axis_index("c")
    sid = lax.axis_index("s")
    wid = sid * NC + cid
    zero16 = jnp.zeros((16,), _f32)

    def z_body(j, c2):
        nacc[pl.ds(j * 16, 16)] = zero16
        return c2
    lax.fori_loop(0, N_PAD // 16, z_body, 0)

    def body(ci, carry):
        base = wid * PER_W + ci * SC_CH
        pltpu.sync_copy(ch_hbm.at[pl.ds(base, SC_CH)], vals)
        pltpu.sync_copy(ec_hbm.at[pl.ds(base, SC_CH)], idxs)

        def inner(j, c2):
            sl = pl.ds(j * 16, 16)
            iv = idxs[sl]
            vv = vals[sl]
            plsc.addupdate_scatter(nacc, [iv], vv)
            return c2
        lax.fori_loop(0, SC_CH // 16, inner, 0)
        return carry

    lax.fori_loop(0, PER_W // SC_CH, body, 0)
    pltpu.sync_copy(nacc, parts_hbm.at[wid])


def _scatter(charges, ec):
    mesh = plsc.VectorSubcoreMesh(core_axis_name="c", subcore_axis_name="s")
    f = functools.partial(
        pl.kernel,
        out_type=jax.ShapeDtypeStruct((NW, N_PAD), _f32),
        mesh=mesh,
        compiler_params=pltpu.CompilerParams(needs_layout_passes=False),
        scratch_types=[
            pltpu.VMEM((SC_CH,), _f32),
            pltpu.VMEM((SC_CH,), jnp.int32),
            pltpu.VMEM((N_PAD,), _f32),
            pltpu.SemaphoreType.DMA,
        ],
    )(_scatter_body)
    return f(charges, ec)


def _reduce_body(parts_ref, batch_ref, atom_ref, tot_ref):
    acc = jnp.zeros((NROW, 128), _f32)
    for w in range(NW):
        acc = acc + parts_ref[w]
    atom_ref[...] = acc
    b = batch_ref[...]
    row_iota = lax.broadcasted_iota(jnp.int32, (G, 1), 0)
    tot = jnp.zeros((G, 1), _f32)
    for g in range(G):
        s = jnp.sum(jnp.where(b == g, acc, 0.0))
        tot = tot + jnp.where(row_iota == g, s, 0.0)
    tot_ref[...] = tot


def _reduce(parts, batch2d):
    return pl.pallas_call(
        _reduce_body,
        out_shape=[
            jax.ShapeDtypeStruct((NROW, 128), _f32),
            jax.ShapeDtypeStruct((G, 1), _f32),
        ],
    )(parts, batch2d)


def kernel(node_attrs, edge_index, edge_embedding, edge_lengths, pos, batch,
           W1, b1, W2, b2, W3, b3):
    ec = edge_index[0]
    en = edge_index[1]
    w1a = W1[:D]
    w1b = W1[D:2 * D]
    w1e = W1[2 * D:]

    pa, pb = _node_proj(node_attrs, w1a, w1b)
    x = _edge_gather(pa, pb, ec, en)
    charges = _edge_mlp(x, edge_embedding, w1e, b1.reshape(1, H), W2,
                        b2.reshape(1, H), W3, b3.reshape(1, 1))
    parts = _scatter(charges.reshape(E), ec).reshape(NW, NROW, 128)
    batch2d = jnp.pad(batch, (0, N_PAD - N)).reshape(NROW, 128)
    atom2d, total = _reduce(parts, batch2d)
    atomic = atom2d.reshape(N_PAD)[:N].reshape(N, 1)
    return atomic, total
